# baseline (speedup 1.0000x reference)
import functools

import jax
import jax.numpy as jnp
import numpy as np
from jax import lax
from jax.experimental import pallas as pl
from jax.experimental.pallas import tpu as pltpu
from jax.experimental.pallas import tpu_sc as plsc

_LABEL_SMOOTHING = 0.1
_V = 100000
_B = 1024
_IGNORE = 1
_S = np.float32(_LABEL_SMOOTHING / (_V - 2))
_C = np.float32(1.0 - _LABEL_SMOOTHING)
_K = float(_V - 2) * float(np.float32(_S * np.float32(np.log(_S)))) + float(
    np.float32(_C * np.float32(np.log(_C)))
)

_BC = 2048
_NB = -(-_V // _BC)
_LW = 128

_NW = 32
_BPW = _B // _NW
_L = 16


_sc_mesh = plsc.VectorSubcoreMesh(core_axis_name="c", subcore_axis_name="s")


@functools.partial(
    pl.kernel,
    mesh=_sc_mesh,
    out_type=jax.ShapeDtypeStruct((_NW, _L), jnp.float32),
    scratch_types=[
        pltpu.VMEM((_BPW,), jnp.int32),
        pltpu.VMEM((2 * _BPW,), jnp.int32),
        pltpu.VMEM((2 * _BPW,), jnp.float32),
        pltpu.VMEM((_L,), jnp.float32),
        pltpu.SemaphoreType.DMA,
    ],
)
def _sc_gather(flat_hbm, tgt_hbm, out_hbm, tgt_v, idx_v, val_v, p_v, sem):
    wid = lax.axis_index("c") * 16 + lax.axis_index("s")
    base = wid * _BPW
    pltpu.sync_copy(tgt_hbm.at[pl.ds(base, _BPW)], tgt_v)
    lanes = lax.iota(jnp.int32, _L)
    for c in range(_BPW // _L):
        row_v = (base + c * _L + lanes) * _V
        t16 = tgt_v[pl.ds(c * _L, _L)]
        idx_v[pl.ds(c * _L, _L)] = row_v + t16
        idx_v[pl.ds(_BPW + c * _L, _L)] = row_v + _IGNORE
    pltpu.async_copy(flat_hbm.at[idx_v], val_v, sem).wait()
    p = jnp.zeros((_L,), jnp.float32)
    for c in range(_BPW // _L):
        t16 = tgt_v[pl.ds(c * _L, _L)]
        g16 = val_v[pl.ds(c * _L, _L)]
        x116 = val_v[pl.ds(_BPW + c * _L, _L)]
        maskf = jnp.where(t16 != _IGNORE, 1.0, 0.0).astype(jnp.float32)
        p = p + maskf * (_K + _S * x116 - (_C - _S) * g16)
    p_v[...] = p
    pltpu.sync_copy(p_v, out_hbm.at[wid])


def _tc_body(x_ref, t_ref, sc_ref, o_ref, acc_ref):
    j = pl.program_id(0)

    @pl.when(j == 0)
    def _init():
        acc_ref[...] = jnp.zeros_like(acc_ref)

    def _accum(x):
        part = x[:, 0:_LW]
        for k in range(1, _BC // _LW):
            part = part + x[:, k * _LW : (k + 1) * _LW]
        acc_ref[...] += part

    @pl.when(j < _NB - 1)
    def _full():
        _accum(x_ref[...])

    @pl.when(j == _NB - 1)
    def _last():
        cols = j * _BC + lax.broadcasted_iota(jnp.int32, (_B, _BC), 1)
        _accum(jnp.where(cols < _V, x_ref[...], 0.0))
        rowsum = jnp.sum(acc_ref[...], axis=1, keepdims=True)
        maskf = (t_ref[...] != _IGNORE).astype(jnp.float32)
        o_ref[...] = (
            jnp.sum(sc_ref[...]) - _S * jnp.sum(maskf * rowsum)
        ).reshape(1, 1)


@functools.partial(jax.jit)
def kernel(output, target):
    t32 = target.astype(jnp.int32)
    sc_part = _sc_gather(output.reshape(_B * _V), t32)
    res = pl.pallas_call(
        _tc_body,
        grid=(_NB,),
        in_specs=[
            pl.BlockSpec((_B, _BC), lambda j: (0, j)),
            pl.BlockSpec((_B, 1), lambda j: (0, 0)),
            pl.BlockSpec((_NW, _L), lambda j: (0, 0)),
        ],
        out_specs=pl.BlockSpec((1, 1), lambda j: (0, 0)),
        out_shape=jax.ShapeDtypeStruct((1, 1), jnp.float32),
        scratch_shapes=[pltpu.VMEM((_B, _LW), jnp.float32)],
    )(output, t32.reshape(_B, 1), sc_part)
    return res[0, 0]


# speedup vs baseline: 1.9154x; 1.9154x over previous
import functools

import jax
import jax.numpy as jnp
import numpy as np
from jax import lax
from jax.experimental import pallas as pl
from jax.experimental.pallas import tpu as pltpu

_LABEL_SMOOTHING = 0.1
_V = 100000
_B = 1024
_IGNORE = 1
_S = np.float32(_LABEL_SMOOTHING / (_V - 2))
_C = np.float32(1.0 - _LABEL_SMOOTHING)
_K = float(_V - 2) * float(np.float32(_S * np.float32(np.log(_S)))) + float(
    np.float32(_C * np.float32(np.log(_C)))
)

_BC = 2048
_NB = -(-_V // _BC)
_LW = 128


_RPB = 8


def _gather_body(t_sref, *refs):
    x_refs = refs[:_RPB]
    x1_ref = refs[_RPB]
    o_ref = refs[_RPB + 1]
    j = pl.program_id(0)

    @pl.when(j == 0)
    def _init():
        o_ref[...] = jnp.zeros_like(o_ref)

    rows = lax.broadcasted_iota(jnp.int32, (_RPB, 128), 0)
    lanes = lax.broadcasted_iota(jnp.int32, (_RPB, 128), 1)
    x1v = jnp.sum(
        jnp.where(lanes == _IGNORE, x1_ref[...], 0.0), axis=1, keepdims=True
    )
    total = jnp.zeros((), jnp.float32)
    for k in range(_RPB):
        tk = t_sref[j * _RPB + k]
        g_k = jnp.sum(
            jnp.where(
                jnp.logical_and(rows == k, lanes == lax.rem(tk, 128)),
                x_refs[k][...],
                0.0,
            )
        )
        x1_k = jnp.sum(jnp.where(rows == k, x1v, 0.0))
        total += jnp.where(
            tk != _IGNORE, _K + _S * x1_k - (_C - _S) * g_k, 0.0
        )
    o_ref[...] += total.reshape(1, 1)


def _rowsum_body(x_ref, t_ref, g_ref, o_ref, acc_ref):
    j = pl.program_id(0)

    @pl.when(j == 0)
    def _init():
        acc_ref[...] = jnp.zeros_like(acc_ref)

    def _accum(x):
        part = x[:, 0:_LW]
        for k in range(1, _BC // _LW):
            part = part + x[:, k * _LW : (k + 1) * _LW]
        acc_ref[...] += part

    @pl.when(j < _NB - 1)
    def _full():
        _accum(x_ref[...])

    @pl.when(j == _NB - 1)
    def _last():
        cols = j * _BC + lax.broadcasted_iota(jnp.int32, (_B, _BC), 1)
        _accum(jnp.where(cols < _V, x_ref[...], 0.0))
        rowsum = jnp.sum(acc_ref[...], axis=1, keepdims=True)
        maskf = (t_ref[...] != _IGNORE).astype(jnp.float32)
        o_ref[...] = (g_ref[...] - _S * jnp.sum(maskf * rowsum)).reshape(1, 1)


@functools.partial(jax.jit)
def kernel(output, target):
    t32 = target.astype(jnp.int32)
    def _mk_spec(k):
        return pl.BlockSpec(
            (_RPB, 128), lambda j, t, k=k: (j, t[j * _RPB + k] // 128)
        )

    gathered = pl.pallas_call(
        _gather_body,
        grid_spec=pltpu.PrefetchScalarGridSpec(
            num_scalar_prefetch=1,
            grid=(_B // _RPB,),
            in_specs=[_mk_spec(k) for k in range(_RPB)]
            + [pl.BlockSpec((_RPB, 128), lambda j, t: (j, 0))],
            out_specs=pl.BlockSpec((1, 1), lambda j, t: (0, 0)),
        ),
        out_shape=jax.ShapeDtypeStruct((1, 1), jnp.float32),
    )(t32, *([output] * _RPB), output)
    res = pl.pallas_call(
        _rowsum_body,
        grid=(_NB,),
        in_specs=[
            pl.BlockSpec((_B, _BC), lambda j: (0, j)),
            pl.BlockSpec((_B, 1), lambda j: (0, 0)),
            pl.BlockSpec((1, 1), lambda j: (0, 0)),
        ],
        out_specs=pl.BlockSpec((1, 1), lambda j: (0, 0)),
        out_shape=jax.ShapeDtypeStruct((1, 1), jnp.float32),
        scratch_shapes=[pltpu.VMEM((_B, _LW), jnp.float32)],
    )(output, t32.reshape(_B, 1), gathered)
    return res[0, 0]


# speedup vs baseline: 8.6702x; 4.5266x over previous
import functools

import jax
import jax.numpy as jnp
import numpy as np
from jax import lax
from jax.experimental import pallas as pl
from jax.experimental.pallas import tpu as pltpu

_LABEL_SMOOTHING = 0.1
_V = 100000
_B = 1024
_IGNORE = 1
_S = np.float32(_LABEL_SMOOTHING / (_V - 2))
_C = np.float32(1.0 - _LABEL_SMOOTHING)
_K = float(_V - 2) * float(np.float32(_S * np.float32(np.log(_S)))) + float(
    np.float32(_C * np.float32(np.log(_C)))
)

_BR = 2048
_NBF = _V // _BR
_TW = _V - _NBF * _BR
_NB = _NBF + 1

_GB = 32
_NBATCH = _B // _GB


def _body(t_sref, x_any, tv_ref, o_ref, blk_ref, gbuf_ref, acc_ref, x1_ref,
          blksem, gsem):
    j = pl.program_id(0)

    def _blk_copy(jj, rows):
        s = lax.rem(jj, 2)
        start = pl.multiple_of(jj * _BR, _BR)
        return pltpu.make_async_copy(
            x_any.at[pl.ds(start, rows), :],
            blk_ref.at[pl.ds(s * _BR, rows), :],
            blksem.at[s],
        )

    def _gdma(b, k):
        t = t_sref[b * _GB + k]
        vstart = pl.multiple_of((t // 8) * 8, 8)
        lstart = pl.multiple_of((b // 4) * 128, 128)
        off = lax.rem(b, 2) * _GB * 8 + k * 8
        return pltpu.make_async_copy(
            x_any.at[pl.ds(vstart, 8), pl.ds(lstart, 128)],
            gbuf_ref.at[pl.ds(off, 8), :],
            gsem.at[lax.rem(b, 2) * _GB + k],
        )

    @pl.when(j == 0)
    def _init():
        o_ref[...] = jnp.zeros_like(o_ref)
        acc_ref[...] = jnp.zeros_like(acc_ref)
        _blk_copy(0, _BR).start()

    @pl.when(j + 1 < _NBF)
    def _prefetch_full():
        _blk_copy(j + 1, _BR).start()

    @pl.when(j + 1 == _NBF)
    def _prefetch_tail():
        _blk_copy(j + 1, _TW).start()

    @pl.when(j < _NBATCH)
    def _gfire():
        for k in range(_GB):
            _gdma(j, k).start()

    @pl.when(jnp.logical_and(j >= 1, j <= _NBATCH))
    def _gdrain():
        b = j - 1
        for k in range(_GB):
            _gdma(b, k).wait()
        rows8 = lax.broadcasted_iota(jnp.int32, (8, 128), 0)
        lanes = lax.broadcasted_iota(jnp.int32, (8, 128), 1)
        part = jnp.zeros((8, 128), jnp.float32)
        for k in range(_GB):
            t = t_sref[b * _GB + k]
            off = lax.rem(b, 2) * _GB * 8 + k * 8
            xw = gbuf_ref[pl.ds(off, 8), :]
            sel = jnp.logical_and(
                jnp.logical_and(rows8 == lax.rem(t, 8),
                                lanes == lax.rem(b, 4) * _GB + k),
                t != _IGNORE,
            )
            part = part + jnp.where(sel, xw, 0.0)
        o_ref[...] += (-(_C - _S) * jnp.sum(part)).reshape(1, 1)

    def _accum(rows):
        s = lax.rem(j, 2)
        xblk = blk_ref[pl.ds(s * _BR, rows), :]
        part = xblk[0:8, :]
        for k in range(1, rows // 8):
            part = part + xblk[k * 8 : (k + 1) * 8, :]
        acc_ref[...] += part
        return xblk

    @pl.when(j < _NBF)
    def _stream_full():
        _blk_copy(j, _BR).wait()
        xblk = _accum(_BR)

        @pl.when(j == 0)
        def _grab_x1():
            x1_ref[...] = xblk[_IGNORE : _IGNORE + 1, :]

    @pl.when(j == _NBF)
    def _stream_tail():
        _blk_copy(j, _TW).wait()
        _accum(_TW)
        rowsum = jnp.sum(acc_ref[...], axis=0, keepdims=True)
        maskf = (tv_ref[...] != _IGNORE).astype(jnp.float32)
        o_ref[...] += (
            _K * jnp.sum(maskf)
            + _S * jnp.sum(maskf * (x1_ref[...] - rowsum))
        ).reshape(1, 1)


@functools.partial(jax.jit)
def kernel(output, target):
    t32 = target.astype(jnp.int32)
    res = pl.pallas_call(
        _body,
        grid_spec=pltpu.PrefetchScalarGridSpec(
            num_scalar_prefetch=1,
            grid=(_NB,),
            in_specs=[
                pl.BlockSpec(memory_space=pl.ANY),
                pl.BlockSpec((1, _B), lambda j, t: (0, 0)),
            ],
            out_specs=pl.BlockSpec((1, 1), lambda j, t: (0, 0)),
            scratch_shapes=[
                pltpu.VMEM((2 * _BR, _B), jnp.float32),
                pltpu.VMEM((2 * _GB * 8, 128), jnp.float32),
                pltpu.VMEM((8, _B), jnp.float32),
                pltpu.VMEM((1, _B), jnp.float32),
                pltpu.SemaphoreType.DMA((2,)),
                pltpu.SemaphoreType.DMA((2 * _GB,)),
            ],
        ),
        out_shape=jax.ShapeDtypeStruct((1, 1), jnp.float32),
    )(t32, jnp.swapaxes(output, 0, 1), t32.reshape(1, _B))
    return res[0, 0]


# speedup vs baseline: 8.6720x; 1.0002x over previous
import functools

import jax
import jax.numpy as jnp
import numpy as np
from jax import lax
from jax.experimental import pallas as pl
from jax.experimental.pallas import tpu as pltpu

_LABEL_SMOOTHING = 0.1
_V = 100000
_B = 1024
_IGNORE = 1
_S = np.float32(_LABEL_SMOOTHING / (_V - 2))
_C = np.float32(1.0 - _LABEL_SMOOTHING)
_K = float(_V - 2) * float(np.float32(_S * np.float32(np.log(_S)))) + float(
    np.float32(_C * np.float32(np.log(_C)))
)

_BR = 2048
_NBF = _V // _BR
_TW = _V - _NBF * _BR
_NB = _NBF + 1

_NBUF = 3
_GB = 32
_NBATCH = _B // _GB


def _body(t_sref, x_any, tv_ref, o_ref, blk_ref, gbuf_ref, acc_ref, x1_ref,
          blksem, gsem):
    j = pl.program_id(0)

    def _blk_copy(jj, rows):
        s = lax.rem(jj, _NBUF)
        start = pl.multiple_of(jj * _BR, _BR)
        return pltpu.make_async_copy(
            x_any.at[pl.ds(start, rows), :],
            blk_ref.at[pl.ds(s * _BR, rows), :],
            blksem.at[s],
        )

    def _gdma(b, k):
        t = t_sref[b * _GB + k]
        vstart = pl.multiple_of((t // 8) * 8, 8)
        lstart = pl.multiple_of((b // 4) * 128, 128)
        off = lax.rem(b, 2) * _GB * 8 + k * 8
        return pltpu.make_async_copy(
            x_any.at[pl.ds(vstart, 8), pl.ds(lstart, 128)],
            gbuf_ref.at[pl.ds(off, 8), :],
            gsem.at[lax.rem(b, 2) * _GB + k],
        )

    @pl.when(j == 0)
    def _init():
        o_ref[...] = jnp.zeros_like(o_ref)
        acc_ref[...] = jnp.zeros_like(acc_ref)
        _blk_copy(0, _BR).start()
        _blk_copy(1, _BR).start()

    @pl.when(j + 2 < _NBF)
    def _prefetch_full():
        _blk_copy(j + 2, _BR).start()

    @pl.when(j + 2 == _NBF)
    def _prefetch_tail():
        _blk_copy(j + 2, _TW).start()

    @pl.when(j < _NBATCH)
    def _gfire():
        for k in range(_GB):
            _gdma(j, k).start()

    @pl.when(jnp.logical_and(j >= 1, j <= _NBATCH))
    def _gdrain():
        b = j - 1
        for k in range(_GB):
            _gdma(b, k).wait()
        rows8 = lax.broadcasted_iota(jnp.int32, (8, 128), 0)
        lanes = lax.broadcasted_iota(jnp.int32, (8, 128), 1)
        part = jnp.zeros((8, 128), jnp.float32)
        for k in range(_GB):
            t = t_sref[b * _GB + k]
            off = lax.rem(b, 2) * _GB * 8 + k * 8
            xw = gbuf_ref[pl.ds(off, 8), :]
            sel = jnp.logical_and(
                jnp.logical_and(rows8 == lax.rem(t, 8),
                                lanes == lax.rem(b, 4) * _GB + k),
                t != _IGNORE,
            )
            part = part + jnp.where(sel, xw, 0.0)
        o_ref[...] += (-(_C - _S) * jnp.sum(part)).reshape(1, 1)

    def _accum(rows):
        s = lax.rem(j, _NBUF)
        xblk = blk_ref[pl.ds(s * _BR, rows), :]
        part = xblk[0:8, :]
        for k in range(1, rows // 8):
            part = part + xblk[k * 8 : (k + 1) * 8, :]
        acc_ref[...] += part
        return xblk

    @pl.when(j < _NBF)
    def _stream_full():
        _blk_copy(j, _BR).wait()
        xblk = _accum(_BR)

        @pl.when(j == 0)
        def _grab_x1():
            x1_ref[...] = xblk[_IGNORE : _IGNORE + 1, :]

    @pl.when(j == _NBF)
    def _stream_tail():
        _blk_copy(j, _TW).wait()
        _accum(_TW)
        rowsum = jnp.sum(acc_ref[...], axis=0, keepdims=True)
        maskf = (tv_ref[...] != _IGNORE).astype(jnp.float32)
        o_ref[...] += (
            _K * jnp.sum(maskf)
            + _S * jnp.sum(maskf * (x1_ref[...] - rowsum))
        ).reshape(1, 1)


@functools.partial(jax.jit)
def kernel(output, target):
    t32 = target.astype(jnp.int32)
    res = pl.pallas_call(
        _body,
        grid_spec=pltpu.PrefetchScalarGridSpec(
            num_scalar_prefetch=1,
            grid=(_NB,),
            in_specs=[
                pl.BlockSpec(memory_space=pl.ANY),
                pl.BlockSpec((1, _B), lambda j, t: (0, 0)),
            ],
            out_specs=pl.BlockSpec((1, 1), lambda j, t: (0, 0)),
            scratch_shapes=[
                pltpu.VMEM((_NBUF * _BR, _B), jnp.float32),
                pltpu.VMEM((2 * _GB * 8, 128), jnp.float32),
                pltpu.VMEM((8, _B), jnp.float32),
                pltpu.VMEM((1, _B), jnp.float32),
                pltpu.SemaphoreType.DMA((_NBUF,)),
                pltpu.SemaphoreType.DMA((2 * _GB,)),
            ],
        ),
        out_shape=jax.ShapeDtypeStruct((1, 1), jnp.float32),
    )(t32, jnp.swapaxes(output, 0, 1), t32.reshape(1, _B))
    return res[0, 0]
